# baseline (speedup 1.0000x reference)
import functools

import jax
import jax.numpy as jnp
from jax import lax
from jax.experimental import pallas as pl
from jax.experimental.pallas import tpu as pltpu
from jax.experimental.pallas import tpu_sc as plsc

N = 100000
DIN = 21
DPAD = 32
H = 64
G = 512
EPS = 1e-5

BM = 2000
NBLK = N // BM


def _scale_shift(st, gamma, beta):
    mean = st[0:1, :] / N
    var = st[1:2, :] / N - mean * mean
    scale = gamma * jax.lax.rsqrt(var + EPS)
    shift = beta - mean * scale
    return scale, shift


def _agg_mm_kernel(x_ref, a_ref, w_ref, b_ref, y_ref, st_ref):
    h = x_ref[...] + a_ref[...]
    y = jnp.dot(h, w_ref[...], preferred_element_type=jnp.float32) + b_ref[...]
    y_ref[...] = y

    @pl.when(pl.program_id(0) == 0)
    def _():
        st_ref[...] = jnp.zeros_like(st_ref)

    st_ref[0:1, :] += jnp.sum(y, axis=0, keepdims=True)
    st_ref[1:2, :] += jnp.sum(y * y, axis=0, keepdims=True)


def _bn_relu_mm_kernel(y_ref, st_ref, g_ref, be_ref, w_ref, b_ref,
                       z_ref, zst_ref):
    scale, shift = _scale_shift(st_ref[...], g_ref[...], be_ref[...])
    t = jnp.maximum(y_ref[...] * scale + shift, 0.0)
    z = jnp.maximum(
        jnp.dot(t, w_ref[...], preferred_element_type=jnp.float32) + b_ref[...],
        0.0)
    z_ref[...] = z

    @pl.when(pl.program_id(0) == 0)
    def _():
        zst_ref[...] = jnp.zeros_like(zst_ref)

    zst_ref[0:1, :] += jnp.sum(z, axis=0, keepdims=True)
    zst_ref[1:2, :] += jnp.sum(z * z, axis=0, keepdims=True)


def _bn_relu_kernel(y_ref, st_ref, g_ref, be_ref, z_ref):
    scale, shift = _scale_shift(st_ref[...], g_ref[...], be_ref[...])
    z_ref[...] = jnp.maximum(y_ref[...] * scale + shift, 0.0)


def _row_spec(width):
    return pl.BlockSpec((BM, width), lambda i: (i, 0))


def _full_spec(shape):
    return pl.BlockSpec(shape, lambda i: tuple(0 for _ in shape))


def _agg_mm(x, agg, w, b):
    width = x.shape[1]
    return pl.pallas_call(
        _agg_mm_kernel,
        grid=(NBLK,),
        in_specs=[_row_spec(width), _row_spec(width),
                  _full_spec((width, H)), _full_spec((1, H))],
        out_specs=[_row_spec(H), _full_spec((2, H))],
        out_shape=[jax.ShapeDtypeStruct((N, H), jnp.float32),
                   jax.ShapeDtypeStruct((2, H), jnp.float32)],
    )(x, agg, w, b.reshape(1, H))


def _bn_relu_mm(y, st, g, be, w, b):
    return pl.pallas_call(
        _bn_relu_mm_kernel,
        grid=(NBLK,),
        in_specs=[_row_spec(H), _full_spec((2, H)), _full_spec((1, H)),
                  _full_spec((1, H)), _full_spec((H, H)), _full_spec((1, H))],
        out_specs=[_row_spec(H), _full_spec((2, H))],
        out_shape=[jax.ShapeDtypeStruct((N, H), jnp.float32),
                   jax.ShapeDtypeStruct((2, H), jnp.float32)],
    )(y, st, g.reshape(1, H), be.reshape(1, H), w, b.reshape(1, H))


def _bn_relu(y, st, g, be):
    return pl.pallas_call(
        _bn_relu_kernel,
        grid=(NBLK,),
        in_specs=[_row_spec(H), _full_spec((2, H)), _full_spec((1, H)),
                  _full_spec((1, H))],
        out_specs=_row_spec(H),
        out_shape=jax.ShapeDtypeStruct((N, H), jnp.float32),
    )(y, st, g.reshape(1, H), be.reshape(1, H))


E = 1600000
NSUB = 16
CE = 2000
ESUB = E // NSUB
NCH = ESUB // CE
NVEC = CE // 16
FW = 64
NBUF = 4
SBROWS = (CE + NBUF * FW + FW - 1) // FW

ZR = 32


def _make_agg(D, R, npass):
    RB = R + 16
    PSZ = RB // NSUB
    PER = R // NSUB
    mesh = plsc.VectorSubcoreMesh(core_axis_name="c", subcore_axis_name="s")

    @functools.partial(
        pl.kernel, mesh=mesh,
        out_type=jax.ShapeDtypeStruct((N, D), jnp.float32),
        compiler_params=pltpu.CompilerParams(needs_layout_passes=False, use_tc_tiling_on_sc=False),
        scratch_types=[
            pltpu.VMEM_SHARED((RB, D), jnp.float32),
            pltpu.VMEM((ZR, D), jnp.float32),
            pltpu.VMEM((CE,), jnp.int32),
            pltpu.VMEM((CE,), jnp.int32),
            pltpu.VMEM((SBROWS, FW), jnp.int32),
            pltpu.VMEM((SBROWS, FW), jnp.int32),
            *[pltpu.VMEM((FW, D), jnp.float32) for _ in range(NBUF)],
            *[pltpu.SemaphoreType.DMA for _ in range(2 * NBUF)],
        ],
    )
    def agg_kernel(src_hbm, dst_hbm, feat_hbm, out_hbm,
                   acc, zbuf, srcc, dstc, csrc, cdst,
                   r0, r1, r2, r3, g0, g1, g2, g3, s0, s1, s2, s3):
        rows = (r0, r1, r2, r3)
        gsem = (g0, g1, g2, g3)
        ssem = (s0, s1, s2, s3)
        c = lax.axis_index("c")
        s = lax.axis_index("s")
        zero16f = jnp.zeros((16,), jnp.float32)
        zero16i = jnp.zeros((16,), jnp.int32)

        def zrow(i, carry):
            for k in range(D // 16):
                zbuf[i, pl.ds(k * 16, 16)] = zero16f
            return carry
        lax.fori_loop(0, ZR, zrow, 0)

        for p in range(npass):
            lo = (c * npass + p) * R
            zoff = s * PSZ
            zfull, zrem = divmod(PSZ, ZR)

            def zfill(j, carry):
                pltpu.sync_copy(zbuf, acc.at[pl.ds(zoff + j * ZR, ZR)])
                return carry

            lax.fori_loop(0, zfull, zfill, 0)
            if zrem:
                pltpu.sync_copy(zbuf.at[pl.ds(0, zrem)],
                                acc.at[pl.ds(zoff + zfull * ZR, zrem)])
            plsc.subcore_barrier()

            def chunk_body(ch, carry):
                off = s * ESUB + ch * CE
                pltpu.sync_copy(src_hbm.at[pl.ds(off, CE)], srcc)
                pltpu.sync_copy(dst_hbm.at[pl.ds(off, CE)], dstc)

                def vec_body(i, base):
                    sv = srcc[pl.ds(i * 16, 16)]
                    dv = dstc[pl.ds(i * 16, 16)]
                    dl = dv - lo
                    m = (dl >= 0) & (dl < R)
                    mi = jnp.where(m, 1, 0)
                    pos = base + plsc.cumsum(mi) - mi
                    pr = lax.shift_right_logical(pos, 6)
                    pc = lax.bitwise_and(pos, 63)
                    plsc.store_scatter(csrc, [pr, pc], sv, mask=m)
                    plsc.store_scatter(cdst, [pr, pc], dl, mask=m)
                    return base + plsc.all_reduce_population_count(m)

                base = lax.fori_loop(0, NVEC, vec_body, zero16i)
                cnt = jnp.max(base)
                iot = lax.iota(jnp.int32, 16)
                dummy = jnp.full((16,), R, jnp.int32)
                for j in range(NBUF * FW // 16):
                    pp = cnt + j * 16 + iot
                    pr = lax.shift_right_logical(pp, 6)
                    pc = lax.bitwise_and(pp, 63)
                    plsc.store_scatter(csrc, [pr, pc], zero16i)
                    plsc.store_scatter(cdst, [pr, pc], dummy)

                ngroups = (cnt + NBUF * FW - 1) // (NBUF * FW)

                def group(gi, carry2):
                    for b in range(NBUF):
                        @pl.when(gi > 0)
                        def _(b=b):
                            pltpu.make_async_copy(
                                rows[b], acc.at[cdst.at[0]],
                                ssem[b]).wait()
                        pltpu.async_copy(
                            feat_hbm.at[csrc.at[NBUF * gi + b]],
                            rows[b], gsem[b])
                    for b in range(NBUF):
                        pltpu.make_async_copy(
                            feat_hbm.at[csrc.at[0]], rows[b],
                            gsem[b]).wait()
                        pltpu.async_copy(rows[b],
                                         acc.at[cdst.at[NBUF * gi + b]],
                                         ssem[b], add=True)
                    return carry2

                lax.fori_loop(0, ngroups, group, 0)

                @pl.when(ngroups > 0)
                def _():
                    for b in range(NBUF):
                        pltpu.make_async_copy(
                            rows[b], acc.at[cdst.at[0]], ssem[b]).wait()
                return carry

            lax.fori_loop(0, NCH, chunk_body, 0)
            plsc.subcore_barrier()

            gbase = lo + s * PER
            nj = jnp.clip(N - gbase, 0, PER) // 160

            def cp(j, carry):
                o = j * 160
                pltpu.sync_copy(acc.at[pl.ds(s * PER + o, 160)],
                                out_hbm.at[pl.ds(gbase + o, 160)])
                return carry

            lax.fori_loop(0, nj, cp, 0)
            plsc.subcore_barrier()

    return agg_kernel


_agg32 = _make_agg(32, 51200, 1)
_agg64 = _make_agg(64, 25600, 2)


NW = 2 * NSUB
PROWS = 3200
PCH = 160


def _make_pool():
    mesh = plsc.VectorSubcoreMesh(core_axis_name="c", subcore_axis_name="s")

    @functools.partial(
        pl.kernel, mesh=mesh,
        out_type=[jax.ShapeDtypeStruct((NW, G, 64), jnp.float32),
                  jax.ShapeDtypeStruct((NW, G, 80), jnp.float32)],
        compiler_params=pltpu.CompilerParams(needs_layout_passes=False,
                                             use_tc_tiling_on_sc=False),
        scratch_types=[
            pltpu.VMEM((G, 64), jnp.float32),
            pltpu.VMEM((G, 80), jnp.float32),
            pltpu.VMEM((PCH, 64), jnp.float32),
            pltpu.VMEM((PCH,), jnp.int32),
            pltpu.VMEM((64,), jnp.float32),
            pltpu.VMEM((64,), jnp.float32),
        ],
    )
    def pool_kernel(z_hbm, b_hbm, sc_hbm, sh_hbm, pmax_hbm, psum_hbm,
                    pmax_t, psum_t, zc, bc, scv, shv):
        c = lax.axis_index("c")
        s = lax.axis_index("s")
        wid = c * NSUB + s
        pltpu.sync_copy(sc_hbm, scv)
        pltpu.sync_copy(sh_hbm, shv)

        ninf = jnp.full((16,), -jnp.inf, jnp.float32)
        zf = jnp.zeros((16,), jnp.float32)

        def init_row(i, carry):
            for k in range(4):
                pmax_t[i, pl.ds(16 * k, 16)] = ninf
            for k in range(5):
                psum_t[i, pl.ds(16 * k, 16)] = zf
            return carry

        lax.fori_loop(0, G, init_row, 0)

        base = wid * PROWS
        nch = jnp.clip(N - base, 0, PROWS) // PCH
        one0 = jnp.where(lax.iota(jnp.int32, 16) == 0, 1.0, 0.0)

        def chunk(jc, carry):
            off = base + jc * PCH
            pltpu.sync_copy(z_hbm.at[pl.ds(off, PCH)], zc)
            pltpu.sync_copy(b_hbm.at[pl.ds(off, PCH)], bc)

            def row16(i, carry2):
                bv = bc[pl.ds(i * 16, 16)]
                for j in range(16):
                    b = bv[j]
                    r = i * 16 + j
                    for k in range(4):
                        sl = pl.ds(16 * k, 16)
                        v = zc[r, sl] * scv[sl] + shv[sl]
                        pmax_t[b, sl] = jnp.maximum(pmax_t[b, sl], v)
                        psum_t[b, sl] = psum_t[b, sl] + v
                    psum_t[b, pl.ds(64, 16)] = psum_t[b, pl.ds(64, 16)] + one0
                return carry2

            lax.fori_loop(0, PCH // 16, row16, 0)
            return carry

        lax.fori_loop(0, nch, chunk, 0)
        pltpu.sync_copy(pmax_t, pmax_hbm.at[wid])
        pltpu.sync_copy(psum_t, psum_hbm.at[wid])

    return pool_kernel


_pool = _make_pool()


def _scale_shift_kernel(st_ref, g_ref, be_ref, sc_ref, sh_ref):
    scale, shift = _scale_shift(st_ref[...], g_ref[...], be_ref[...])
    sc_ref[...] = scale
    sh_ref[...] = shift


def _final_scale_shift(st, g, be):
    return pl.pallas_call(
        _scale_shift_kernel,
        out_shape=[jax.ShapeDtypeStruct((1, H), jnp.float32),
                   jax.ShapeDtypeStruct((1, H), jnp.float32)],
    )(st, g.reshape(1, H), be.reshape(1, H))


def _pool_merge_kernel(pm_ref, ps_ref, out_ref):
    m = jnp.max(pm_ref[...], axis=0)
    sm = jnp.sum(ps_ref[...], axis=0)
    mean = sm[:, :64] / jnp.maximum(sm[:, 64:65], 1.0)
    out_ref[...] = jnp.concatenate([m, mean], axis=1)


def _pool_merge(pm, ps):
    return pl.pallas_call(
        _pool_merge_kernel,
        out_shape=jax.ShapeDtypeStruct((G, 2 * H), jnp.float32),
    )(pm, ps)


def kernel(x, edge_index, batch, W1a, b1a, g1a, be1a, W1b, b1b,
           W2a, b2a, g2a, be2a, W2b, b2b, go1, beo1, go2, beo2):
    src = edge_index[0]
    dst = edge_index[1]

    x_pad = jnp.pad(x, ((0, 0), (0, DPAD - DIN)))
    w1a_pad = jnp.pad(W1a, ((0, DPAD - DIN), (0, 0)))

    agg1 = _agg32(src, dst, x_pad)
    y1, st_y1 = _agg_mm(x_pad, agg1, w1a_pad, b1a)
    h1, st_h1 = _bn_relu_mm(y1, st_y1, g1a, be1a, W1b, b1b)
    h1n = _bn_relu(h1, st_h1, go1, beo1)

    agg2 = _agg64(src, dst, h1n)
    y2, st_y2 = _agg_mm(h1n, agg2, W2a, b2a)
    z2, st_z2 = _bn_relu_mm(y2, st_y2, g2a, be2a, W2b, b2b)

    sc2, sh2 = _final_scale_shift(st_z2, go2, beo2)
    pm, ps = _pool(z2, batch, sc2.reshape(H), sh2.reshape(H))
    return _pool_merge(pm, ps)


# speedup vs baseline: 1.5837x; 1.5837x over previous
import functools

import jax
import jax.numpy as jnp
from jax import lax
from jax.experimental import pallas as pl
from jax.experimental.pallas import tpu as pltpu
from jax.experimental.pallas import tpu_sc as plsc

N = 100000
DIN = 21
DPAD = 32
H = 64
G = 512
EPS = 1e-5

BM = 2000
NBLK = N // BM


def _scale_shift(st, gamma, beta):
    mean = st[0:1, :] / N
    var = st[1:2, :] / N - mean * mean
    scale = gamma * jax.lax.rsqrt(var + EPS)
    shift = beta - mean * scale
    return scale, shift


def _agg_mm_kernel(x_ref, a_ref, w_ref, b_ref, y_ref, st_ref):
    h = x_ref[...] + a_ref[...]
    y = jnp.dot(h, w_ref[...], preferred_element_type=jnp.float32) + b_ref[...]
    y_ref[...] = y

    @pl.when(pl.program_id(0) == 0)
    def _():
        st_ref[...] = jnp.zeros_like(st_ref)

    st_ref[0:1, :] += jnp.sum(y, axis=0, keepdims=True)
    st_ref[1:2, :] += jnp.sum(y * y, axis=0, keepdims=True)


def _bn_relu_mm_kernel(y_ref, st_ref, g_ref, be_ref, w_ref, b_ref,
                       z_ref, zst_ref):
    scale, shift = _scale_shift(st_ref[...], g_ref[...], be_ref[...])
    t = jnp.maximum(y_ref[...] * scale + shift, 0.0)
    z = jnp.maximum(
        jnp.dot(t, w_ref[...], preferred_element_type=jnp.float32) + b_ref[...],
        0.0)
    z_ref[...] = z

    @pl.when(pl.program_id(0) == 0)
    def _():
        zst_ref[...] = jnp.zeros_like(zst_ref)

    zst_ref[0:1, :] += jnp.sum(z, axis=0, keepdims=True)
    zst_ref[1:2, :] += jnp.sum(z * z, axis=0, keepdims=True)


def _bn_relu_kernel(y_ref, st_ref, g_ref, be_ref, z_ref):
    scale, shift = _scale_shift(st_ref[...], g_ref[...], be_ref[...])
    z_ref[...] = jnp.maximum(y_ref[...] * scale + shift, 0.0)


def _row_spec(width):
    return pl.BlockSpec((BM, width), lambda i: (i, 0))


def _full_spec(shape):
    return pl.BlockSpec(shape, lambda i: tuple(0 for _ in shape))


def _agg_mm(x, agg, w, b):
    width = x.shape[1]
    return pl.pallas_call(
        _agg_mm_kernel,
        grid=(NBLK,),
        in_specs=[_row_spec(width), _row_spec(width),
                  _full_spec((width, H)), _full_spec((1, H))],
        out_specs=[_row_spec(H), _full_spec((2, H))],
        out_shape=[jax.ShapeDtypeStruct((N, H), jnp.float32),
                   jax.ShapeDtypeStruct((2, H), jnp.float32)],
    )(x, agg, w, b.reshape(1, H))


def _bn_relu_mm(y, st, g, be, w, b):
    return pl.pallas_call(
        _bn_relu_mm_kernel,
        grid=(NBLK,),
        in_specs=[_row_spec(H), _full_spec((2, H)), _full_spec((1, H)),
                  _full_spec((1, H)), _full_spec((H, H)), _full_spec((1, H))],
        out_specs=[_row_spec(H), _full_spec((2, H))],
        out_shape=[jax.ShapeDtypeStruct((N, H), jnp.float32),
                   jax.ShapeDtypeStruct((2, H), jnp.float32)],
    )(y, st, g.reshape(1, H), be.reshape(1, H), w, b.reshape(1, H))


def _bn_relu(y, st, g, be):
    return pl.pallas_call(
        _bn_relu_kernel,
        grid=(NBLK,),
        in_specs=[_row_spec(H), _full_spec((2, H)), _full_spec((1, H)),
                  _full_spec((1, H))],
        out_specs=_row_spec(H),
        out_shape=jax.ShapeDtypeStruct((N, H), jnp.float32),
    )(y, st, g.reshape(1, H), be.reshape(1, H))


E = 1600000
NSUB = 16
CE = 2000
ESUB = E // NSUB
NCH = ESUB // CE
NVEC = CE // 16
FW = 64
SBROWS = (CE + 2 * FW + FW - 1) // FW
UNR = 5

ZR = 64


def _make_agg(D, R, npass):
    RB = R + 256
    PSZ = RB // NSUB
    PER = R // NSUB
    mesh = plsc.VectorSubcoreMesh(core_axis_name="c", subcore_axis_name="s")

    @functools.partial(
        pl.kernel, mesh=mesh,
        out_type=jax.ShapeDtypeStruct((N, D), jnp.float32),
        compiler_params=pltpu.CompilerParams(needs_layout_passes=False, use_tc_tiling_on_sc=False),
        scratch_types=[
            pltpu.VMEM_SHARED((RB, D), jnp.float32),
            pltpu.VMEM((ZR, D), jnp.float32),
            pltpu.VMEM((CE,), jnp.int32),
            pltpu.VMEM((CE,), jnp.int32),
            pltpu.VMEM((SBROWS, FW), jnp.int32),
            pltpu.VMEM((SBROWS, FW), jnp.int32),
            pltpu.VMEM((FW, D), jnp.float32),
            pltpu.VMEM((FW, D), jnp.float32),
            pltpu.SemaphoreType.DMA,
            pltpu.SemaphoreType.DMA,
        ],
    )
    def agg_kernel(src_hbm, dst_hbm, feat_hbm, out_hbm,
                   acc, zbuf, srcc, dstc, csrc, cdst, rowsa, rowsb,
                   sema, semb):
        c = lax.axis_index("c")
        s = lax.axis_index("s")
        zero16f = jnp.zeros((16,), jnp.float32)
        zero16i = jnp.zeros((16,), jnp.int32)

        def zrow(i, carry):
            for k in range(D // 16):
                zbuf[i, pl.ds(k * 16, 16)] = zero16f
            return carry
        lax.fori_loop(0, ZR, zrow, 0)

        for p in range(npass):
            lo = (c * npass + p) * R
            zoff = s * PSZ
            zfull, zrem = divmod(PSZ, ZR)

            def zfill(j, carry):
                pltpu.sync_copy(zbuf, acc.at[pl.ds(zoff + j * ZR, ZR)])
                return carry

            lax.fori_loop(0, zfull, zfill, 0)
            if zrem:
                pltpu.sync_copy(zbuf.at[pl.ds(0, zrem)],
                                acc.at[pl.ds(zoff + zfull * ZR, zrem)])
            plsc.subcore_barrier()

            def chunk_body(ch, carry):
                off = s * ESUB + ch * CE
                pltpu.sync_copy(src_hbm.at[pl.ds(off, CE)], srcc)
                pltpu.sync_copy(dst_hbm.at[pl.ds(off, CE)], dstc)

                ones16 = jnp.ones((16,), jnp.int32)

                def vec_body(i, base):
                    for u in range(UNR):
                        sl = pl.ds((i * UNR + u) * 16, 16)
                        sv = srcc[sl]
                        dv = dstc[sl]
                        dl = dv - lo
                        m = (dl >= 0) & (dl < R)
                        inc = plsc.cumsum(ones16, mask=m)
                        pos = base + inc - 1
                        pr = lax.shift_right_logical(pos, 6)
                        pc = lax.bitwise_and(pos, 63)
                        plsc.store_scatter(csrc, [pr, pc], sv, mask=m)
                        plsc.store_scatter(cdst, [pr, pc], dl, mask=m)
                        base = base + plsc.all_reduce_population_count(m)
                    return base

                base = lax.fori_loop(0, NVEC // UNR, vec_body, zero16i)
                cnt = jnp.max(base)
                iot = lax.iota(jnp.int32, 16)
                dummy = jnp.full((16,), R, jnp.int32)
                for j in range(2 * FW // 16):
                    pp = cnt + j * 16 + iot
                    pr = lax.shift_right_logical(pp, 6)
                    pc = lax.bitwise_and(pp, 63)
                    plsc.store_scatter(csrc, [pr, pc], zero16i)
                    plsc.store_scatter(cdst, [pr, pc], dummy)

                npairs = (cnt + 2 * FW - 1) // (2 * FW)

                def fire(j, carry2):
                    ca = pltpu.async_copy(feat_hbm.at[csrc.at[2 * j]],
                                          rowsa, sema)
                    cb = pltpu.async_copy(feat_hbm.at[csrc.at[2 * j + 1]],
                                          rowsb, semb)
                    ca.wait()
                    pltpu.sync_copy(rowsa, acc.at[cdst.at[2 * j]], add=True)
                    cb.wait()
                    pltpu.sync_copy(rowsb, acc.at[cdst.at[2 * j + 1]],
                                    add=True)
                    return carry2

                lax.fori_loop(0, npairs, fire, 0)
                return carry

            lax.fori_loop(0, NCH, chunk_body, 0)
            plsc.subcore_barrier()

            gbase = lo + s * PER
            nj = jnp.clip(N - gbase, 0, PER) // 160

            def cp(j, carry):
                o = j * 160
                pltpu.sync_copy(acc.at[pl.ds(s * PER + o, 160)],
                                out_hbm.at[pl.ds(gbase + o, 160)])
                return carry

            lax.fori_loop(0, nj, cp, 0)
            plsc.subcore_barrier()

    return agg_kernel


_agg32 = _make_agg(32, 51200, 1)
_agg64 = _make_agg(64, 25600, 2)


NW = 2 * NSUB
PROWS = 3200
PCH = 160


def _make_pool():
    mesh = plsc.VectorSubcoreMesh(core_axis_name="c", subcore_axis_name="s")

    @functools.partial(
        pl.kernel, mesh=mesh,
        out_type=[jax.ShapeDtypeStruct((NW, G, 64), jnp.float32),
                  jax.ShapeDtypeStruct((NW, G, 80), jnp.float32)],
        compiler_params=pltpu.CompilerParams(needs_layout_passes=False,
                                             use_tc_tiling_on_sc=False),
        scratch_types=[
            pltpu.VMEM((G, 64), jnp.float32),
            pltpu.VMEM((G, 80), jnp.float32),
            pltpu.VMEM((PCH, 64), jnp.float32),
            pltpu.VMEM((PCH,), jnp.int32),
            pltpu.VMEM((64,), jnp.float32),
            pltpu.VMEM((64,), jnp.float32),
        ],
    )
    def pool_kernel(z_hbm, b_hbm, sc_hbm, sh_hbm, pmax_hbm, psum_hbm,
                    pmax_t, psum_t, zc, bc, scv, shv):
        c = lax.axis_index("c")
        s = lax.axis_index("s")
        wid = c * NSUB + s
        pltpu.sync_copy(sc_hbm, scv)
        pltpu.sync_copy(sh_hbm, shv)

        ninf = jnp.full((16,), -jnp.inf, jnp.float32)
        zf = jnp.zeros((16,), jnp.float32)

        def init_row(i, carry):
            for k in range(4):
                pmax_t[i, pl.ds(16 * k, 16)] = ninf
            for k in range(5):
                psum_t[i, pl.ds(16 * k, 16)] = zf
            return carry

        lax.fori_loop(0, G, init_row, 0)

        base = wid * PROWS
        nch = jnp.clip(N - base, 0, PROWS) // PCH
        one0 = jnp.where(lax.iota(jnp.int32, 16) == 0, 1.0, 0.0)

        def chunk(jc, carry):
            off = base + jc * PCH
            pltpu.sync_copy(z_hbm.at[pl.ds(off, PCH)], zc)
            pltpu.sync_copy(b_hbm.at[pl.ds(off, PCH)], bc)

            def row16(i, carry2):
                bv = bc[pl.ds(i * 16, 16)]
                for j in range(16):
                    b = bv[j]
                    r = i * 16 + j
                    for k in range(4):
                        sl = pl.ds(16 * k, 16)
                        v = zc[r, sl] * scv[sl] + shv[sl]
                        pmax_t[b, sl] = jnp.maximum(pmax_t[b, sl], v)
                        psum_t[b, sl] = psum_t[b, sl] + v
                    psum_t[b, pl.ds(64, 16)] = psum_t[b, pl.ds(64, 16)] + one0
                return carry2

            lax.fori_loop(0, PCH // 16, row16, 0)
            return carry

        lax.fori_loop(0, nch, chunk, 0)
        pltpu.sync_copy(pmax_t, pmax_hbm.at[wid])
        pltpu.sync_copy(psum_t, psum_hbm.at[wid])

    return pool_kernel


_pool = _make_pool()


def _scale_shift_kernel(st_ref, g_ref, be_ref, sc_ref, sh_ref):
    scale, shift = _scale_shift(st_ref[...], g_ref[...], be_ref[...])
    sc_ref[...] = scale
    sh_ref[...] = shift


def _final_scale_shift(st, g, be):
    return pl.pallas_call(
        _scale_shift_kernel,
        out_shape=[jax.ShapeDtypeStruct((1, H), jnp.float32),
                   jax.ShapeDtypeStruct((1, H), jnp.float32)],
    )(st, g.reshape(1, H), be.reshape(1, H))


def _pool_merge_kernel(pm_ref, ps_ref, out_ref):
    m = jnp.max(pm_ref[...], axis=0)
    sm = jnp.sum(ps_ref[...], axis=0)
    mean = sm[:, :64] / jnp.maximum(sm[:, 64:65], 1.0)
    out_ref[...] = jnp.concatenate([m, mean], axis=1)


def _pool_merge(pm, ps):
    return pl.pallas_call(
        _pool_merge_kernel,
        out_shape=jax.ShapeDtypeStruct((G, 2 * H), jnp.float32),
    )(pm, ps)


def kernel(x, edge_index, batch, W1a, b1a, g1a, be1a, W1b, b1b,
           W2a, b2a, g2a, be2a, W2b, b2b, go1, beo1, go2, beo2):
    src = edge_index[0]
    dst = edge_index[1]

    x_pad = jnp.pad(x, ((0, 0), (0, DPAD - DIN)))
    w1a_pad = jnp.pad(W1a, ((0, DPAD - DIN), (0, 0)))

    agg1 = _agg32(src, dst, x_pad)
    y1, st_y1 = _agg_mm(x_pad, agg1, w1a_pad, b1a)
    h1, st_h1 = _bn_relu_mm(y1, st_y1, g1a, be1a, W1b, b1b)
    h1n = _bn_relu(h1, st_h1, go1, beo1)

    agg2 = _agg64(src, dst, h1n)
    y2, st_y2 = _agg_mm(h1n, agg2, W2a, b2a)
    z2, st_z2 = _bn_relu_mm(y2, st_y2, g2a, be2a, W2b, b2b)

    sc2, sh2 = _final_scale_shift(st_z2, go2, beo2)
    pm, ps = _pool(z2, batch, sc2.reshape(H), sh2.reshape(H))
    return _pool_merge(pm, ps)


# speedup vs baseline: 2.3015x; 1.4533x over previous
import functools

import jax
import jax.numpy as jnp
from jax import lax
from jax.experimental import pallas as pl
from jax.experimental.pallas import tpu as pltpu
from jax.experimental.pallas import tpu_sc as plsc

N = 100000
DIN = 21
DPAD = 32
H = 64
G = 512
EPS = 1e-5

BM = 2000
NBLK = N // BM


def _scale_shift(st, gamma, beta):
    mean = st[0:1, :] / N
    var = st[1:2, :] / N - mean * mean
    scale = gamma * jax.lax.rsqrt(var + EPS)
    shift = beta - mean * scale
    return scale, shift


def _agg_mm_kernel(x_ref, a_ref, w_ref, b_ref, y_ref, st_ref):
    h = x_ref[...] + a_ref[...]
    y = jnp.dot(h, w_ref[...], preferred_element_type=jnp.float32) + b_ref[...]
    y_ref[...] = y

    @pl.when(pl.program_id(0) == 0)
    def _():
        st_ref[...] = jnp.zeros_like(st_ref)

    st_ref[0:1, :] += jnp.sum(y, axis=0, keepdims=True)
    st_ref[1:2, :] += jnp.sum(y * y, axis=0, keepdims=True)


def _bn_relu_mm_kernel(y_ref, st_ref, g_ref, be_ref, w_ref, b_ref,
                       z_ref, zst_ref):
    scale, shift = _scale_shift(st_ref[...], g_ref[...], be_ref[...])
    t = jnp.maximum(y_ref[...] * scale + shift, 0.0)
    z = jnp.maximum(
        jnp.dot(t, w_ref[...], preferred_element_type=jnp.float32) + b_ref[...],
        0.0)
    z_ref[...] = z

    @pl.when(pl.program_id(0) == 0)
    def _():
        zst_ref[...] = jnp.zeros_like(zst_ref)

    zst_ref[0:1, :] += jnp.sum(z, axis=0, keepdims=True)
    zst_ref[1:2, :] += jnp.sum(z * z, axis=0, keepdims=True)


def _agg_mm2_kernel(x0_ref, x1_ref, a0_ref, a1_ref, w_ref, b_ref,
                    y_ref, st_ref):
    h = jnp.concatenate([x0_ref[...] + a0_ref[...],
                         x1_ref[...] + a1_ref[...]], axis=1)
    y = jnp.dot(h, w_ref[...], preferred_element_type=jnp.float32) + b_ref[...]
    y_ref[...] = y

    @pl.when(pl.program_id(0) == 0)
    def _():
        st_ref[...] = jnp.zeros_like(st_ref)

    st_ref[0:1, :] += jnp.sum(y, axis=0, keepdims=True)
    st_ref[1:2, :] += jnp.sum(y * y, axis=0, keepdims=True)


def _agg_mm2(x0, x1, a0, a1, w, b):
    return pl.pallas_call(
        _agg_mm2_kernel,
        grid=(NBLK,),
        in_specs=[_row_spec(32), _row_spec(32), _row_spec(32), _row_spec(32),
                  _full_spec((H, H)), _full_spec((1, H))],
        out_specs=[_row_spec(H), _full_spec((2, H))],
        out_shape=[jax.ShapeDtypeStruct((N, H), jnp.float32),
                   jax.ShapeDtypeStruct((2, H), jnp.float32)],
    )(x0, x1, a0, a1, w, b.reshape(1, H))


def _bn_relu_kernel(y_ref, st_ref, g_ref, be_ref, z0_ref, z1_ref):
    scale, shift = _scale_shift(st_ref[...], g_ref[...], be_ref[...])
    z = jnp.maximum(y_ref[...] * scale + shift, 0.0)
    z0_ref[...] = z[:, :32]
    z1_ref[...] = z[:, 32:]


def _row_spec(width):
    return pl.BlockSpec((BM, width), lambda i: (i, 0))


def _full_spec(shape):
    return pl.BlockSpec(shape, lambda i: tuple(0 for _ in shape))


def _agg_mm(x, agg, w, b):
    width = x.shape[1]
    return pl.pallas_call(
        _agg_mm_kernel,
        grid=(NBLK,),
        in_specs=[_row_spec(width), _row_spec(width),
                  _full_spec((width, H)), _full_spec((1, H))],
        out_specs=[_row_spec(H), _full_spec((2, H))],
        out_shape=[jax.ShapeDtypeStruct((N, H), jnp.float32),
                   jax.ShapeDtypeStruct((2, H), jnp.float32)],
    )(x, agg, w, b.reshape(1, H))


def _bn_relu_mm(y, st, g, be, w, b):
    return pl.pallas_call(
        _bn_relu_mm_kernel,
        grid=(NBLK,),
        in_specs=[_row_spec(H), _full_spec((2, H)), _full_spec((1, H)),
                  _full_spec((1, H)), _full_spec((H, H)), _full_spec((1, H))],
        out_specs=[_row_spec(H), _full_spec((2, H))],
        out_shape=[jax.ShapeDtypeStruct((N, H), jnp.float32),
                   jax.ShapeDtypeStruct((2, H), jnp.float32)],
    )(y, st, g.reshape(1, H), be.reshape(1, H), w, b.reshape(1, H))


def _bn_relu(y, st, g, be):
    return pl.pallas_call(
        _bn_relu_kernel,
        grid=(NBLK,),
        in_specs=[_row_spec(H), _full_spec((2, H)), _full_spec((1, H)),
                  _full_spec((1, H))],
        out_specs=[_row_spec(32), _row_spec(32)],
        out_shape=[jax.ShapeDtypeStruct((N, 32), jnp.float32),
                   jax.ShapeDtypeStruct((N, 32), jnp.float32)],
    )(y, st, g.reshape(1, H), be.reshape(1, H))


E = 1600000
NSUB = 16
CE = 2000
ESUB = E // NSUB
NCH = ESUB // CE
NVEC = CE // 16
FW = 64
SBROWS = (CE + 2 * FW + FW - 1) // FW
UNR = 5

ZR = 64


def _make_agg(D, R, npass):
    RB = R + 256
    PSZ = RB // NSUB
    PER = R // NSUB
    mesh = plsc.VectorSubcoreMesh(core_axis_name="c", subcore_axis_name="s")

    @functools.partial(
        pl.kernel, mesh=mesh,
        out_type=jax.ShapeDtypeStruct((N, D), jnp.float32),
        compiler_params=pltpu.CompilerParams(needs_layout_passes=False, use_tc_tiling_on_sc=False),
        scratch_types=[
            pltpu.VMEM_SHARED((RB, D), jnp.float32),
            pltpu.VMEM((ZR, D), jnp.float32),
            pltpu.VMEM((CE,), jnp.int32),
            pltpu.VMEM((CE,), jnp.int32),
            pltpu.VMEM((SBROWS, FW), jnp.int32),
            pltpu.VMEM((SBROWS, FW), jnp.int32),
            pltpu.VMEM((FW, D), jnp.float32),
            pltpu.VMEM((FW, D), jnp.float32),
            pltpu.SemaphoreType.DMA,
            pltpu.SemaphoreType.DMA,
        ],
    )
    def agg_kernel(src_hbm, dst_hbm, feat_hbm, out_hbm,
                   acc, zbuf, srcc, dstc, csrc, cdst, rowsa, rowsb,
                   sema, semb):
        c = lax.axis_index("c")
        s = lax.axis_index("s")
        zero16f = jnp.zeros((16,), jnp.float32)
        zero16i = jnp.zeros((16,), jnp.int32)

        def zrow(i, carry):
            for k in range(D // 16):
                zbuf[i, pl.ds(k * 16, 16)] = zero16f
            return carry
        lax.fori_loop(0, ZR, zrow, 0)

        for p in range(npass):
            lo = (c * npass + p) * R
            zoff = s * PSZ
            zfull, zrem = divmod(PSZ, ZR)

            def zfill(j, carry):
                pltpu.sync_copy(zbuf, acc.at[pl.ds(zoff + j * ZR, ZR)])
                return carry

            lax.fori_loop(0, zfull, zfill, 0)
            if zrem:
                pltpu.sync_copy(zbuf.at[pl.ds(0, zrem)],
                                acc.at[pl.ds(zoff + zfull * ZR, zrem)])
            plsc.subcore_barrier()

            def chunk_body(ch, carry):
                off = s * ESUB + ch * CE
                pltpu.sync_copy(src_hbm.at[pl.ds(off, CE)], srcc)
                pltpu.sync_copy(dst_hbm.at[pl.ds(off, CE)], dstc)

                ones16 = jnp.ones((16,), jnp.int32)

                def vec_body(i, base):
                    for u in range(UNR):
                        sl = pl.ds((i * UNR + u) * 16, 16)
                        sv = srcc[sl]
                        dv = dstc[sl]
                        dl = dv - lo
                        m = (dl >= 0) & (dl < R)
                        inc = plsc.cumsum(ones16, mask=m)
                        pos = base + inc - 1
                        pr = lax.shift_right_logical(pos, 6)
                        pc = lax.bitwise_and(pos, 63)
                        plsc.store_scatter(csrc, [pr, pc], sv, mask=m)
                        plsc.store_scatter(cdst, [pr, pc], dl, mask=m)
                        base = base + plsc.all_reduce_population_count(m)
                    return base

                base = lax.fori_loop(0, NVEC // UNR, vec_body, zero16i)
                cnt = jnp.max(base)
                iot = lax.iota(jnp.int32, 16)
                dummy = jnp.full((16,), R, jnp.int32)
                for j in range(2 * FW // 16):
                    pp = cnt + j * 16 + iot
                    pr = lax.shift_right_logical(pp, 6)
                    pc = lax.bitwise_and(pp, 63)
                    plsc.store_scatter(csrc, [pr, pc], zero16i)
                    plsc.store_scatter(cdst, [pr, pc], dummy)

                npairs = (cnt + 2 * FW - 1) // (2 * FW)

                def fire(j, carry2):
                    ca = pltpu.async_copy(feat_hbm.at[csrc.at[2 * j]],
                                          rowsa, sema)
                    cb = pltpu.async_copy(feat_hbm.at[csrc.at[2 * j + 1]],
                                          rowsb, semb)
                    ca.wait()
                    pltpu.sync_copy(rowsa, acc.at[cdst.at[2 * j]], add=True)
                    cb.wait()
                    pltpu.sync_copy(rowsb, acc.at[cdst.at[2 * j + 1]],
                                    add=True)
                    return carry2

                lax.fori_loop(0, npairs, fire, 0)
                return carry

            lax.fori_loop(0, NCH, chunk_body, 0)
            plsc.subcore_barrier()

            gbase = lo + s * PER
            nj = jnp.clip(N - gbase, 0, PER) // 160

            def cp(j, carry):
                o = j * 160
                pltpu.sync_copy(acc.at[pl.ds(s * PER + o, 160)],
                                out_hbm.at[pl.ds(gbase + o, 160)])
                return carry

            lax.fori_loop(0, nj, cp, 0)
            plsc.subcore_barrier()

    return agg_kernel


_agg32 = _make_agg(32, 51200, 1)


NW = 2 * NSUB
PROWS = 3200
PCH = 160


def _make_pool():
    mesh = plsc.VectorSubcoreMesh(core_axis_name="c", subcore_axis_name="s")

    @functools.partial(
        pl.kernel, mesh=mesh,
        out_type=[jax.ShapeDtypeStruct((NW, G, 64), jnp.float32),
                  jax.ShapeDtypeStruct((NW, G, 80), jnp.float32)],
        compiler_params=pltpu.CompilerParams(needs_layout_passes=False,
                                             use_tc_tiling_on_sc=False),
        scratch_types=[
            pltpu.VMEM((G, 64), jnp.float32),
            pltpu.VMEM((G, 80), jnp.float32),
            pltpu.VMEM((PCH, 64), jnp.float32),
            pltpu.VMEM((PCH,), jnp.int32),
            pltpu.VMEM((64,), jnp.float32),
            pltpu.VMEM((64,), jnp.float32),
        ],
    )
    def pool_kernel(z_hbm, b_hbm, sc_hbm, sh_hbm, pmax_hbm, psum_hbm,
                    pmax_t, psum_t, zc, bc, scv, shv):
        c = lax.axis_index("c")
        s = lax.axis_index("s")
        wid = c * NSUB + s
        pltpu.sync_copy(sc_hbm, scv)
        pltpu.sync_copy(sh_hbm, shv)

        ninf = jnp.full((16,), -jnp.inf, jnp.float32)
        zf = jnp.zeros((16,), jnp.float32)

        def init_row(i, carry):
            for k in range(4):
                pmax_t[i, pl.ds(16 * k, 16)] = ninf
            for k in range(5):
                psum_t[i, pl.ds(16 * k, 16)] = zf
            return carry

        lax.fori_loop(0, G, init_row, 0)

        base = wid * PROWS
        nch = jnp.clip(N - base, 0, PROWS) // PCH
        one0 = jnp.where(lax.iota(jnp.int32, 16) == 0, 1.0, 0.0)

        def chunk(jc, carry):
            off = base + jc * PCH
            pltpu.sync_copy(z_hbm.at[pl.ds(off, PCH)], zc)
            pltpu.sync_copy(b_hbm.at[pl.ds(off, PCH)], bc)

            def row16(i, carry2):
                bv = bc[pl.ds(i * 16, 16)]
                for j in range(16):
                    b = bv[j]
                    r = i * 16 + j
                    for k in range(4):
                        sl = pl.ds(16 * k, 16)
                        v = zc[r, sl] * scv[sl] + shv[sl]
                        pmax_t[b, sl] = jnp.maximum(pmax_t[b, sl], v)
                        psum_t[b, sl] = psum_t[b, sl] + v
                    psum_t[b, pl.ds(64, 16)] = psum_t[b, pl.ds(64, 16)] + one0
                return carry2

            lax.fori_loop(0, PCH // 16, row16, 0)
            return carry

        lax.fori_loop(0, nch, chunk, 0)
        pltpu.sync_copy(pmax_t, pmax_hbm.at[wid])
        pltpu.sync_copy(psum_t, psum_hbm.at[wid])

    return pool_kernel


_pool = _make_pool()


def _scale_shift_kernel(st_ref, g_ref, be_ref, sc_ref, sh_ref):
    scale, shift = _scale_shift(st_ref[...], g_ref[...], be_ref[...])
    sc_ref[...] = scale
    sh_ref[...] = shift


def _final_scale_shift(st, g, be):
    return pl.pallas_call(
        _scale_shift_kernel,
        out_shape=[jax.ShapeDtypeStruct((1, H), jnp.float32),
                   jax.ShapeDtypeStruct((1, H), jnp.float32)],
    )(st, g.reshape(1, H), be.reshape(1, H))


def _pool_merge_kernel(pm_ref, ps_ref, out_ref):
    m = jnp.max(pm_ref[...], axis=0)
    sm = jnp.sum(ps_ref[...], axis=0)
    mean = sm[:, :64] / jnp.maximum(sm[:, 64:65], 1.0)
    out_ref[...] = jnp.concatenate([m, mean], axis=1)


def _pool_merge(pm, ps):
    return pl.pallas_call(
        _pool_merge_kernel,
        out_shape=jax.ShapeDtypeStruct((G, 2 * H), jnp.float32),
    )(pm, ps)


def kernel(x, edge_index, batch, W1a, b1a, g1a, be1a, W1b, b1b,
           W2a, b2a, g2a, be2a, W2b, b2b, go1, beo1, go2, beo2):
    src = edge_index[0]
    dst = edge_index[1]

    x_pad = jnp.pad(x, ((0, 0), (0, DPAD - DIN)))
    w1a_pad = jnp.pad(W1a, ((0, DPAD - DIN), (0, 0)))

    agg1 = _agg32(src, dst, x_pad)
    y1, st_y1 = _agg_mm(x_pad, agg1, w1a_pad, b1a)
    h1, st_h1 = _bn_relu_mm(y1, st_y1, g1a, be1a, W1b, b1b)
    h1n0, h1n1 = _bn_relu(h1, st_h1, go1, beo1)

    agg2a = _agg32(src, dst, h1n0)
    agg2b = _agg32(src, dst, h1n1)
    y2, st_y2 = _agg_mm2(h1n0, h1n1, agg2a, agg2b, W2a, b2a)
    z2, st_z2 = _bn_relu_mm(y2, st_y2, g2a, be2a, W2b, b2b)

    sc2, sh2 = _final_scale_shift(st_z2, go2, beo2)
    pm, ps = _pool(z2, batch, sc2.reshape(H), sh2.reshape(H))
    return _pool_merge(pm, ps)
